# baseline (device time: 68296 ns/iter reference)
import jax
import jax.numpy as jnp
from jax import lax
from jax.experimental import pallas as pl
from jax.experimental.pallas import tpu as pltpu

N_DEV = 8
N_LAYERS = 3


def kernel(x, Win0, Wout0, Win1, Wout1, Win2, Wout2):
    b, d_shard = x.shape
    h_dim = Win0.shape[1]
    chunk = b // N_DEV

    def body(x_ref, win0_ref, wout0_ref, win1_ref, wout1_ref, win2_ref,
             wout2_ref, out_ref, acc_ref, hown_ref, hfull_ref, rs_buf,
             rs_send, rs_recv, ag_send, ag_recv):
        my = lax.axis_index("i")
        wins = (win0_ref, win1_ref, win2_ref)
        wouts = (wout0_ref, wout1_ref, wout2_ref)

        out_ref[...] = x_ref[...]

        def recv_desc(dst, sem_arr, s):
            return pltpu.make_async_remote_copy(
                src_ref=dst, dst_ref=dst,
                send_sem=sem_arr.at[s], recv_sem=sem_arr.at[s],
                device_id=(0,), device_id_type=pl.DeviceIdType.MESH,
            )

        prev_rs = []
        prev_ag = []
        for l in range(N_LAYERS):
            for r in prev_rs:
                r.wait_send()
            rs_rdmas = []
            for j in range(N_DEV):
                acc_ref[j, :, :] = jnp.dot(
                    out_ref[pl.ds(j * chunk, chunk), :], wins[l][...],
                    preferred_element_type=jnp.float32,
                )
                rdma = pltpu.make_async_remote_copy(
                    src_ref=acc_ref.at[j],
                    dst_ref=rs_buf.at[my],
                    send_sem=rs_send.at[j],
                    recv_sem=rs_recv.at[my],
                    device_id=(j,),
                    device_id_type=pl.DeviceIdType.MESH,
                )
                rdma.start()
                rs_rdmas.append(rdma)
            prev_rs = rs_rdmas

            h = None
            for s in range(N_DEV):
                recv_desc(rs_buf.at[s], rs_recv, s).wait_recv()
                h = rs_buf[s] if h is None else h + rs_buf[s]
            h = jnp.maximum(h, 0.0)

            for r in prev_ag:
                r.wait_send()
            hown_ref[...] = h

            ag_rdmas = []
            for j in range(N_DEV):
                rdma = pltpu.make_async_remote_copy(
                    src_ref=hown_ref,
                    dst_ref=hfull_ref.at[my],
                    send_sem=ag_send.at[j],
                    recv_sem=ag_recv.at[my],
                    device_id=(j,),
                    device_id_type=pl.DeviceIdType.MESH,
                )
                rdma.start()
                ag_rdmas.append(rdma)
            prev_ag = ag_rdmas

            for s in range(N_DEV):
                recv_desc(hfull_ref.at[s], ag_recv, s).wait_recv()
                out_ref[pl.ds(s * chunk, chunk), :] = jnp.dot(
                    hfull_ref[s], wouts[l][...],
                    preferred_element_type=jnp.float32,
                )

        for r in prev_rs:
            r.wait_send()
        for r in prev_ag:
            r.wait_send()

    return pl.pallas_call(
        body,
        out_shape=jax.ShapeDtypeStruct((b, d_shard), jnp.float32),
        in_specs=[pl.BlockSpec(memory_space=pltpu.VMEM)] * 7,
        out_specs=pl.BlockSpec(memory_space=pltpu.VMEM),
        scratch_shapes=[
            pltpu.VMEM((N_DEV, chunk, h_dim), jnp.float32),
            pltpu.VMEM((chunk, h_dim), jnp.float32),
            pltpu.VMEM((N_DEV, chunk, h_dim), jnp.float32),
            pltpu.VMEM((N_DEV, chunk, h_dim), jnp.float32),
            pltpu.SemaphoreType.DMA((N_DEV,)),
            pltpu.SemaphoreType.DMA((N_DEV,)),
            pltpu.SemaphoreType.DMA((N_DEV,)),
            pltpu.SemaphoreType.DMA((N_DEV,)),
        ],
    )(x, Win0, Wout0, Win1, Wout1, Win2, Wout2)


# device time: 67249 ns/iter; 1.0156x vs baseline; 1.0156x over previous
import jax
import jax.numpy as jnp
from jax import lax
from jax.experimental import pallas as pl
from jax.experimental.pallas import tpu as pltpu

N_DEV = 8
N_LAYERS = 3
K_FAR_FIRST = (6, 2, 5, 7, 1, 3, 4)
K_NEAR_FIRST = (1, 3, 4, 2, 5, 7, 6)


def kernel(x, Win0, Wout0, Win1, Wout1, Win2, Wout2):
    b, d_shard = x.shape
    h_dim = Win0.shape[1]
    chunk = b // N_DEV

    def body(x_ref, win0_ref, wout0_ref, win1_ref, wout1_ref, win2_ref,
             wout2_ref, out_ref, acc_ref, hown_ref, hfull_ref, rs_buf,
             rs_send, rs_recv, ag_send, ag_recv):
        my = lax.axis_index("i")
        wins = (win0_ref, win1_ref, win2_ref)
        wouts = (wout0_ref, wout1_ref, wout2_ref)

        out_ref[...] = x_ref[...]

        prev_rs = []
        prev_ag = []
        for l in range(N_LAYERS):
            for r in prev_rs:
                r.wait_send()
            rs_rdmas = {}
            for k in K_FAR_FIRST:
                d = my ^ k
                acc_ref[k, :, :] = jnp.dot(
                    out_ref[pl.ds(d * chunk, chunk), :], wins[l][...],
                    preferred_element_type=jnp.float32,
                )
                rdma = pltpu.make_async_remote_copy(
                    src_ref=acc_ref.at[k],
                    dst_ref=rs_buf.at[k],
                    send_sem=rs_send.at[k],
                    recv_sem=rs_recv.at[k],
                    device_id=(d,),
                    device_id_type=pl.DeviceIdType.MESH,
                )
                rdma.start()
                rs_rdmas[k] = rdma
            prev_rs = list(rs_rdmas.values())
            acc_ref[0, :, :] = jnp.dot(
                out_ref[pl.ds(my * chunk, chunk), :], wins[l][...],
                preferred_element_type=jnp.float32,
            )

            h = acc_ref[0]
            for k in K_NEAR_FIRST:
                rs_rdmas[k].wait_recv()
                h = h + rs_buf[k]
            h = jnp.maximum(h, 0.0)

            for r in prev_ag:
                r.wait_send()
            hown_ref[...] = h

            ag_rdmas = {}
            for k in K_FAR_FIRST:
                d = my ^ k
                rdma = pltpu.make_async_remote_copy(
                    src_ref=hown_ref,
                    dst_ref=hfull_ref.at[k],
                    send_sem=ag_send.at[k],
                    recv_sem=ag_recv.at[k],
                    device_id=(d,),
                    device_id_type=pl.DeviceIdType.MESH,
                )
                rdma.start()
                ag_rdmas[k] = rdma
            prev_ag = list(ag_rdmas.values())

            out_ref[pl.ds(my * chunk, chunk), :] = jnp.dot(
                hown_ref[...], wouts[l][...],
                preferred_element_type=jnp.float32,
            )
            for k in K_NEAR_FIRST:
                d = my ^ k
                ag_rdmas[k].wait_recv()
                out_ref[pl.ds(d * chunk, chunk), :] = jnp.dot(
                    hfull_ref[k], wouts[l][...],
                    preferred_element_type=jnp.float32,
                )

        for r in prev_rs:
            r.wait_send()
        for r in prev_ag:
            r.wait_send()

    return pl.pallas_call(
        body,
        out_shape=jax.ShapeDtypeStruct((b, d_shard), jnp.float32),
        in_specs=[pl.BlockSpec(memory_space=pltpu.VMEM)] * 7,
        out_specs=pl.BlockSpec(memory_space=pltpu.VMEM),
        scratch_shapes=[
            pltpu.VMEM((N_DEV, chunk, h_dim), jnp.float32),
            pltpu.VMEM((chunk, h_dim), jnp.float32),
            pltpu.VMEM((N_DEV, chunk, h_dim), jnp.float32),
            pltpu.VMEM((N_DEV, chunk, h_dim), jnp.float32),
            pltpu.SemaphoreType.DMA((N_DEV,)),
            pltpu.SemaphoreType.DMA((N_DEV,)),
            pltpu.SemaphoreType.DMA((N_DEV,)),
            pltpu.SemaphoreType.DMA((N_DEV,)),
        ],
    )(x, Win0, Wout0, Win1, Wout1, Win2, Wout2)


# device time: 63962 ns/iter; 1.0678x vs baseline; 1.0514x over previous
import jax
import jax.numpy as jnp
from jax import lax
from jax.experimental import pallas as pl
from jax.experimental.pallas import tpu as pltpu

N_DEV = 8
N_LAYERS = 3
K_FAR_FIRST = (6, 2, 5, 7, 1, 3, 4)
K_NEAR_FIRST = (1, 3, 4, 2, 5, 7, 6)


def kernel(x, Win0, Wout0, Win1, Wout1, Win2, Wout2):
    b, d_shard = x.shape
    h_dim = Win0.shape[1]
    chunk = b // N_DEV

    def body(x_ref, win0_ref, wout0_ref, win1_ref, wout1_ref, win2_ref,
             wout2_ref, out_ref, acc_ref, hown_ref, hfull_ref, rs_buf,
             rs_send, rs_recv, ag_send, ag_recv, loc_sems):
        my = lax.axis_index("i")
        wins = (win0_ref, win1_ref, win2_ref)
        wouts = (wout0_ref, wout1_ref, wout2_ref)

        out_ref[...] = x_ref[...]

        prev_rs = []
        prev_ag = []
        for l in range(N_LAYERS):
            with jax.named_scope(f"mm1#l={l}"):
                for r in prev_rs:
                    r.wait_send()
                acc_ref[...] = jnp.dot(
                    out_ref[...], wins[l][...],
                    preferred_element_type=jnp.float32,
                ).reshape(N_DEV, chunk, h_dim)

            with jax.named_scope(f"rs_send#l={l}"):
                rs_rdmas = {}
                for k in K_FAR_FIRST:
                    d = my ^ k
                    rdma = pltpu.make_async_remote_copy(
                        src_ref=acc_ref.at[d],
                        dst_ref=rs_buf.at[k],
                        send_sem=rs_send.at[k],
                        recv_sem=rs_recv.at[k],
                        device_id=(d,),
                        device_id_type=pl.DeviceIdType.MESH,
                    )
                    rdma.start()
                    rs_rdmas[k] = rdma
                prev_rs = list(rs_rdmas.values())
                own = pltpu.make_async_copy(
                    acc_ref.at[my], rs_buf.at[0], loc_sems.at[0],
                )
                own.start()
                own.wait()

            with jax.named_scope(f"rs_wait_sum#l={l}"):
                h = rs_buf[0]
                for k in K_NEAR_FIRST:
                    rs_rdmas[k].wait_recv()
                    h = h + rs_buf[k]
                h = jnp.maximum(h, 0.0)
                for r in prev_ag:
                    r.wait_send()
                hown_ref[...] = h

            with jax.named_scope(f"ag_send#l={l}"):
                ag_rdmas = {}
                for k in K_FAR_FIRST:
                    d = my ^ k
                    rdma = pltpu.make_async_remote_copy(
                        src_ref=hown_ref,
                        dst_ref=hfull_ref.at[my],
                        send_sem=ag_send.at[k],
                        recv_sem=ag_recv.at[k],
                        device_id=(d,),
                        device_id_type=pl.DeviceIdType.MESH,
                    )
                    rdma.start()
                    ag_rdmas[k] = rdma
                prev_ag = list(ag_rdmas.values())
                own = pltpu.make_async_copy(
                    hown_ref, hfull_ref.at[my], loc_sems.at[1],
                )
                own.start()
                own.wait()

            with jax.named_scope(f"ag_wait#l={l}"):
                for k in K_NEAR_FIRST:
                    ag_rdmas[k].wait_recv()

            with jax.named_scope(f"mm2#l={l}"):
                out_ref[...] = jnp.dot(
                    hfull_ref[...].reshape(b, h_dim), wouts[l][...],
                    preferred_element_type=jnp.float32,
                )

        for r in prev_rs:
            r.wait_send()
        for r in prev_ag:
            r.wait_send()

    return pl.pallas_call(
        body,
        out_shape=jax.ShapeDtypeStruct((b, d_shard), jnp.float32),
        in_specs=[pl.BlockSpec(memory_space=pltpu.VMEM)] * 7,
        out_specs=pl.BlockSpec(memory_space=pltpu.VMEM),
        scratch_shapes=[
            pltpu.VMEM((N_DEV, chunk, h_dim), jnp.float32),
            pltpu.VMEM((chunk, h_dim), jnp.float32),
            pltpu.VMEM((N_DEV, chunk, h_dim), jnp.float32),
            pltpu.VMEM((N_DEV, chunk, h_dim), jnp.float32),
            pltpu.SemaphoreType.DMA((N_DEV,)),
            pltpu.SemaphoreType.DMA((N_DEV,)),
            pltpu.SemaphoreType.DMA((N_DEV,)),
            pltpu.SemaphoreType.DMA((N_DEV,)),
            pltpu.SemaphoreType.DMA((2,)),
        ],
    )(x, Win0, Wout0, Win1, Wout1, Win2, Wout2)


# device time: 46883 ns/iter; 1.4567x vs baseline; 1.3643x over previous
import jax
import jax.numpy as jnp
from jax import lax
from jax.experimental import pallas as pl
from jax.experimental.pallas import tpu as pltpu

N_DEV = 8
N_LAYERS = 3
K_FAR_FIRST = (6, 2, 5, 7, 1, 3, 4)
K_NEAR_FIRST = (1, 3, 4, 2, 5, 7, 6)


def kernel(x, Win0, Wout0, Win1, Wout1, Win2, Wout2):
    b, d_shard = x.shape
    h_dim = Win0.shape[1]
    chunk = b // N_DEV
    bf16 = jnp.bfloat16

    def body(x_ref, win0_ref, wout0_ref, win1_ref, wout1_ref, win2_ref,
             wout2_ref, out_ref, acc_ref, hown_ref, hfull_ref, rs_buf,
             rs_send, rs_recv, ag_send, ag_recv, loc_sems):
        my = lax.axis_index("i")
        wins = (win0_ref, win1_ref, win2_ref)
        wouts = (wout0_ref, wout1_ref, wout2_ref)

        out_ref[...] = x_ref[...]

        prev_rs = []
        prev_ag = []
        for l in range(N_LAYERS):
            for r in prev_rs:
                r.wait_send()
            acc_ref[...] = jnp.dot(
                out_ref[...], wins[l][...],
                preferred_element_type=jnp.float32,
            ).astype(bf16).reshape(N_DEV, chunk, h_dim)

            rs_rdmas = {}
            for k in K_FAR_FIRST:
                d = my ^ k
                rdma = pltpu.make_async_remote_copy(
                    src_ref=acc_ref.at[d],
                    dst_ref=rs_buf.at[k],
                    send_sem=rs_send.at[k],
                    recv_sem=rs_recv.at[k],
                    device_id=(d,),
                    device_id_type=pl.DeviceIdType.MESH,
                )
                rdma.start()
                rs_rdmas[k] = rdma
            prev_rs = list(rs_rdmas.values())
            own = pltpu.make_async_copy(
                acc_ref.at[my], rs_buf.at[0], loc_sems.at[0],
            )
            own.start()
            own.wait()

            h = rs_buf[0].astype(jnp.float32)
            for k in K_NEAR_FIRST:
                rs_rdmas[k].wait_recv()
                h = h + rs_buf[k].astype(jnp.float32)
            h = jnp.maximum(h, 0.0)

            for r in prev_ag:
                r.wait_send()
            hown_ref[...] = h.astype(bf16)

            ag_rdmas = {}
            for k in K_FAR_FIRST:
                d = my ^ k
                rdma = pltpu.make_async_remote_copy(
                    src_ref=hown_ref,
                    dst_ref=hfull_ref.at[my],
                    send_sem=ag_send.at[k],
                    recv_sem=ag_recv.at[k],
                    device_id=(d,),
                    device_id_type=pl.DeviceIdType.MESH,
                )
                rdma.start()
                ag_rdmas[k] = rdma
            prev_ag = list(ag_rdmas.values())
            own = pltpu.make_async_copy(
                hown_ref, hfull_ref.at[my], loc_sems.at[1],
            )
            own.start()
            own.wait()

            for k in K_NEAR_FIRST:
                ag_rdmas[k].wait_recv()

            out_ref[...] = jnp.dot(
                hfull_ref[...].reshape(b, h_dim), wouts[l][...],
                preferred_element_type=jnp.float32,
            )

        for r in prev_rs:
            r.wait_send()
        for r in prev_ag:
            r.wait_send()

    return pl.pallas_call(
        body,
        out_shape=jax.ShapeDtypeStruct((b, d_shard), jnp.float32),
        in_specs=[pl.BlockSpec(memory_space=pltpu.VMEM)] * 7,
        out_specs=pl.BlockSpec(memory_space=pltpu.VMEM),
        scratch_shapes=[
            pltpu.VMEM((N_DEV, chunk, h_dim), bf16),
            pltpu.VMEM((chunk, h_dim), bf16),
            pltpu.VMEM((N_DEV, chunk, h_dim), bf16),
            pltpu.VMEM((N_DEV, chunk, h_dim), bf16),
            pltpu.SemaphoreType.DMA((N_DEV,)),
            pltpu.SemaphoreType.DMA((N_DEV,)),
            pltpu.SemaphoreType.DMA((N_DEV,)),
            pltpu.SemaphoreType.DMA((N_DEV,)),
            pltpu.SemaphoreType.DMA((2,)),
        ],
    )(x, Win0, Wout0, Win1, Wout1, Win2, Wout2)


# device time: 46800 ns/iter; 1.4593x vs baseline; 1.0018x over previous
import jax
import jax.numpy as jnp
from jax import lax
from jax.experimental import pallas as pl
from jax.experimental.pallas import tpu as pltpu

N_DEV = 8
N_LAYERS = 3
K_FAR_FIRST = (6, 2, 5, 7, 1, 3, 4)
K_NEAR_FIRST = (1, 3, 4, 2, 5, 7, 6)


def kernel(x, Win0, Wout0, Win1, Wout1, Win2, Wout2):
    b, d_shard = x.shape
    h_dim = Win0.shape[1]
    chunk = b // N_DEV
    bf16 = jnp.bfloat16

    def body(x_ref, win0_ref, wout0_ref, win1_ref, wout1_ref, win2_ref,
             wout2_ref, out_ref, acc_ref, hown_ref, hfull_ref, rs_buf,
             rs_send, rs_recv, ag_send, ag_recv):
        my = lax.axis_index("i")
        wins = (win0_ref, win1_ref, win2_ref)
        wouts = (wout0_ref, wout1_ref, wout2_ref)

        prev_rs = []
        prev_ag = []
        for l in range(N_LAYERS):
            for r in prev_rs:
                r.wait_send()
            xin = x_ref[...] if l == 0 else out_ref[...]
            acc_ref[...] = jnp.dot(
                xin, wins[l][...],
                preferred_element_type=jnp.float32,
            ).astype(bf16).reshape(N_DEV, chunk, h_dim)

            rs_rdmas = {}
            for k in K_FAR_FIRST:
                d = my ^ k
                rdma = pltpu.make_async_remote_copy(
                    src_ref=acc_ref.at[d],
                    dst_ref=rs_buf.at[k],
                    send_sem=rs_send.at[k],
                    recv_sem=rs_recv.at[k],
                    device_id=(d,),
                    device_id_type=pl.DeviceIdType.MESH,
                )
                rdma.start()
                rs_rdmas[k] = rdma
            prev_rs = list(rs_rdmas.values())

            h = acc_ref[my].astype(jnp.float32)
            for k in K_NEAR_FIRST:
                rs_rdmas[k].wait_recv()
                h = h + rs_buf[k].astype(jnp.float32)
            h = jnp.maximum(h, 0.0)

            for r in prev_ag:
                r.wait_send()
            h16 = h.astype(bf16)
            hown_ref[...] = h16
            hfull_ref[my, :, :] = h16

            ag_rdmas = {}
            for k in K_FAR_FIRST:
                d = my ^ k
                rdma = pltpu.make_async_remote_copy(
                    src_ref=hown_ref,
                    dst_ref=hfull_ref.at[my],
                    send_sem=ag_send.at[k],
                    recv_sem=ag_recv.at[k],
                    device_id=(d,),
                    device_id_type=pl.DeviceIdType.MESH,
                )
                rdma.start()
                ag_rdmas[k] = rdma
            prev_ag = list(ag_rdmas.values())

            for k in K_NEAR_FIRST:
                ag_rdmas[k].wait_recv()

            out_ref[...] = jnp.dot(
                hfull_ref[...].reshape(b, h_dim), wouts[l][...],
                preferred_element_type=jnp.float32,
            )

        for r in prev_rs:
            r.wait_send()
        for r in prev_ag:
            r.wait_send()

    return pl.pallas_call(
        body,
        out_shape=jax.ShapeDtypeStruct((b, d_shard), jnp.float32),
        in_specs=[pl.BlockSpec(memory_space=pltpu.VMEM)] * 7,
        out_specs=pl.BlockSpec(memory_space=pltpu.VMEM),
        scratch_shapes=[
            pltpu.VMEM((N_DEV, chunk, h_dim), bf16),
            pltpu.VMEM((chunk, h_dim), bf16),
            pltpu.VMEM((N_DEV, chunk, h_dim), bf16),
            pltpu.VMEM((N_DEV, chunk, h_dim), bf16),
            pltpu.SemaphoreType.DMA((N_DEV,)),
            pltpu.SemaphoreType.DMA((N_DEV,)),
            pltpu.SemaphoreType.DMA((N_DEV,)),
            pltpu.SemaphoreType.DMA((N_DEV,)),
        ],
    )(x, Win0, Wout0, Win1, Wout1, Win2, Wout2)
